# baseline (speedup 1.0000x reference)
import functools

import jax
import jax.numpy as jnp
import numpy as np
from jax import lax
from jax.experimental import pallas as pl
from jax.experimental.pallas import tpu as pltpu
from jax.experimental.pallas import tpu_sc as plsc

RHO_NUM = 100
B = 16384
D = 11

_DGW = 128
_ZERO_COL = 127
_IND_COL = 126
_TC_BLK = 512
_TC_GRID = B // _TC_BLK
_ROWS_PER_W = B // 32
_GROUPS = _ROWS_PER_W // 16


def _bucket_thresholds():
    c = np.float32(1.0 / RHO_NUM)
    t = np.array([np.float32(0.01 * i) for i in range(1, RHO_NUM)], np.float32)

    def bucket_id(s):
        r = np.sqrt(np.float32(s), dtype=np.float32)
        k = np.round(np.float32(r / c)).astype(np.float32)
        return int(np.sum(np.float32(k * c) >= t))

    out = np.empty(RHO_NUM - 1, np.float32)
    for j in range(1, RHO_NUM):
        lo, hi = 0, int(np.array(1e8, np.float32).view(np.uint32))
        while lo < hi:
            mid = (lo + hi) // 2
            if bucket_id(np.array(mid, np.uint32).view(np.float32)) >= j:
                hi = mid
            else:
                lo = mid + 1
        out[j - 1] = np.array(lo, np.uint32).view(np.float32)
    return out


_S_LIST = [float(v) for v in _bucket_thresholds()]


def _tc_body(z1_ref, z2_ref, dg_ref, dgp_ref, gidx_ref, s_scr, m_scr):
    i = pl.program_id(0)
    dx = z1_ref[:, D - 2] - z2_ref[:, D - 2]
    dy = z1_ref[:, D - 1] - z2_ref[:, D - 1]
    s = dx * dx + dy * dy
    mask = (z1_ref[:, 0] > 0.0) & (z2_ref[:, 0] > 0.0)
    sl = _TC_BLK // 128
    s_scr[...] = s.reshape(sl, 128)
    m_scr[...] = jnp.where(mask, 1.0, 0.0).reshape(sl, 128)
    s8 = s_scr[...]
    mask8 = m_scr[...] > 0.0
    bid8 = jnp.zeros((sl, 128), jnp.int32)
    for thr in _S_LIST:
        bid8 = bid8 + (s8 >= thr).astype(jnp.int32)
    col8 = jnp.where(mask8, bid8, _ZERO_COL)
    n8 = (lax.broadcasted_iota(jnp.int32, (sl, 128), 0) * 128
          + lax.broadcasted_iota(jnp.int32, (sl, 128), 1))
    gidx_ref[...] = ((i * _TC_BLK + n8) * _DGW + col8).reshape(1, sl, 128)

    ne = jnp.max(jnp.where(z1_ref[...] != z2_ref[...], 1.0, 0.0))
    dgp_ref[:, :RHO_NUM] = dg_ref[...]
    dgp_ref[:, RHO_NUM:] = jnp.zeros((_TC_BLK, _DGW - RHO_NUM), jnp.float32)
    dgp_ref[:, _IND_COL:_IND_COL + 1] = jnp.full((_TC_BLK, 1), ne,
                                                 jnp.float32)


def _tc_stage(z_1, z_2, dist_grade):
    return pl.pallas_call(
        _tc_body,
        grid=(_TC_GRID,),
        in_specs=[
            pl.BlockSpec((_TC_BLK, D), lambda i: (i, 0)),
            pl.BlockSpec((_TC_BLK, D), lambda i: (i, 0)),
            pl.BlockSpec((_TC_BLK, RHO_NUM), lambda i: (i, 0)),
        ],
        out_specs=[
            pl.BlockSpec((_TC_BLK, _DGW), lambda i: (i, 0)),
            pl.BlockSpec((1, _TC_BLK // 128, 128), lambda i: (i, 0, 0)),
        ],
        out_shape=[
            jax.ShapeDtypeStruct((B, _DGW), jnp.float32),
            jax.ShapeDtypeStruct((_TC_GRID, _TC_BLK // 128, 128), jnp.int32),
        ],
        scratch_shapes=[
            pltpu.VMEM((_TC_BLK // 128, 128), jnp.float32),
            pltpu.VMEM((_TC_BLK // 128, 128), jnp.float32),
        ],
    )(z_1, z_2, dist_grade)


def _sc_body(dgp_hbm, gidx_hbm, sat_hbm, idxv, valv, iidxv, indv, sem):
    wid = lax.axis_index("s") * 2 + lax.axis_index("c")
    rbase = wid * _ROWS_PER_W

    pltpu.sync_copy(gidx_hbm.at[pl.ds(rbase, _ROWS_PER_W)], idxv)
    iidxv[...] = lax.iota(jnp.int32, 16) * (_TC_BLK * _DGW) + _IND_COL

    copies = [
        pltpu.async_copy(
            dgp_hbm.at[idxv.at[pl.ds(i * 128, 128)]],
            valv.at[pl.ds(i * 128, 128)],
            sem,
        )
        for i in range(_ROWS_PER_W // 128)
    ]
    copies.append(pltpu.async_copy(dgp_hbm.at[iidxv], indv, sem))
    for c in copies:
        c.wait()

    gate = jnp.where(jnp.max(indv[...]) > 0.0, 1.0, 0.0)
    for g in range(_GROUPS):
        sl = pl.ds(g * 16, 16)
        valv[sl] = valv[sl] * gate
    pltpu.sync_copy(valv, sat_hbm.at[pl.ds(rbase, _ROWS_PER_W)])


_sc_fn = functools.partial(
    pl.kernel,
    mesh=plsc.VectorSubcoreMesh(core_axis_name="c", subcore_axis_name="s"),
    compiler_params=pltpu.CompilerParams(
        needs_layout_passes=False,
        skip_device_barrier=True,
        disable_semaphore_checks=True,
    ),
    out_type=jax.ShapeDtypeStruct((B,), jnp.float32),
    scratch_types=[
        pltpu.VMEM((_ROWS_PER_W,), jnp.int32),
        pltpu.VMEM((_ROWS_PER_W,), jnp.float32),
        pltpu.VMEM((16,), jnp.int32),
        pltpu.VMEM((16,), jnp.float32),
        pltpu.SemaphoreType.DMA,
    ],
)(_sc_body)


def kernel(z_1, z_2, dist_grade, img, given_param):
    dgp, gidx = _tc_stage(z_1, z_2, dist_grade)
    return _sc_fn(dgp.reshape(-1), gidx.reshape(-1))


# speedup vs baseline: 1.1758x; 1.1758x over previous
import functools

import jax
import jax.numpy as jnp
import numpy as np
from jax import lax
from jax.experimental import pallas as pl
from jax.experimental.pallas import tpu as pltpu
from jax.experimental.pallas import tpu_sc as plsc

RHO_NUM = 100
B = 16384
D = 11

_DGW = 128
_ZERO_COL = 127
_IND_COL = 126
_TC_BLK = 2048
_TC_GRID = B // _TC_BLK
_ROWS_PER_W = B // 32
_GROUPS = _ROWS_PER_W // 16


def _bucket_thresholds():
    c = np.float32(1.0 / RHO_NUM)
    t = np.array([np.float32(0.01 * i) for i in range(1, RHO_NUM)], np.float32)

    def bucket_id(s):
        r = np.sqrt(np.float32(s), dtype=np.float32)
        k = np.round(np.float32(r / c)).astype(np.float32)
        return int(np.sum(np.float32(k * c) >= t))

    out = np.empty(RHO_NUM - 1, np.float32)
    for j in range(1, RHO_NUM):
        lo, hi = 0, int(np.array(1e8, np.float32).view(np.uint32))
        while lo < hi:
            mid = (lo + hi) // 2
            if bucket_id(np.array(mid, np.uint32).view(np.float32)) >= j:
                hi = mid
            else:
                lo = mid + 1
        out[j - 1] = np.array(lo, np.uint32).view(np.float32)
    return out


_S_LIST = [float(v) for v in _bucket_thresholds()]


def _tc_body(z1_ref, z2_ref, dg_ref, dgp_ref, gidx_ref, s_scr, m_scr):
    i = pl.program_id(0)
    dx = z1_ref[:, D - 2] - z2_ref[:, D - 2]
    dy = z1_ref[:, D - 1] - z2_ref[:, D - 1]
    s = dx * dx + dy * dy
    mask = (z1_ref[:, 0] > 0.0) & (z2_ref[:, 0] > 0.0)
    sl = _TC_BLK // 128
    s_scr[...] = s.reshape(sl, 128)
    m_scr[...] = jnp.where(mask, 1.0, 0.0).reshape(sl, 128)
    s8 = s_scr[...]
    mask8 = m_scr[...] > 0.0
    bid8 = jnp.zeros((sl, 128), jnp.int32)
    for thr in _S_LIST:
        bid8 = bid8 + (s8 >= thr).astype(jnp.int32)
    col8 = jnp.where(mask8, bid8, _ZERO_COL)
    n8 = (lax.broadcasted_iota(jnp.int32, (sl, 128), 0) * 128
          + lax.broadcasted_iota(jnp.int32, (sl, 128), 1))
    gidx_ref[...] = ((i * _TC_BLK + n8) * _DGW + col8).reshape(1, sl, 128)

    ne = jnp.max(jnp.where(z1_ref[...] != z2_ref[...], 1.0, 0.0))
    dgp_ref[:, :RHO_NUM] = dg_ref[...]
    dgp_ref[:, RHO_NUM:] = jnp.zeros((_TC_BLK, _DGW - RHO_NUM), jnp.float32)
    dgp_ref[:, _IND_COL:_IND_COL + 1] = jnp.full((_TC_BLK, 1), ne,
                                                 jnp.float32)


def _tc_stage(z_1, z_2, dist_grade):
    return pl.pallas_call(
        _tc_body,
        grid=(_TC_GRID,),
        in_specs=[
            pl.BlockSpec((_TC_BLK, D), lambda i: (i, 0)),
            pl.BlockSpec((_TC_BLK, D), lambda i: (i, 0)),
            pl.BlockSpec((_TC_BLK, RHO_NUM), lambda i: (i, 0)),
        ],
        out_specs=[
            pl.BlockSpec((_TC_BLK, _DGW), lambda i: (i, 0)),
            pl.BlockSpec((1, _TC_BLK // 128, 128), lambda i: (i, 0, 0)),
        ],
        out_shape=[
            jax.ShapeDtypeStruct((B, _DGW), jnp.float32),
            jax.ShapeDtypeStruct((_TC_GRID, _TC_BLK // 128, 128), jnp.int32),
        ],
        scratch_shapes=[
            pltpu.VMEM((_TC_BLK // 128, 128), jnp.float32),
            pltpu.VMEM((_TC_BLK // 128, 128), jnp.float32),
        ],
    )(z_1, z_2, dist_grade)


def _sc_body(dgp_hbm, gidx_hbm, sat_hbm, idxv, valv, iidxv, indv, sem):
    wid = lax.axis_index("s") * 2 + lax.axis_index("c")
    rbase = wid * _ROWS_PER_W

    pltpu.sync_copy(gidx_hbm.at[pl.ds(rbase, _ROWS_PER_W)], idxv)
    iidxv[...] = (jnp.minimum(lax.iota(jnp.int32, 16), _TC_GRID - 1)
                  * (_TC_BLK * _DGW) + _IND_COL)

    copies = [
        pltpu.async_copy(
            dgp_hbm.at[idxv.at[pl.ds(i * 128, 128)]],
            valv.at[pl.ds(i * 128, 128)],
            sem,
        )
        for i in range(_ROWS_PER_W // 128)
    ]
    copies.append(pltpu.async_copy(dgp_hbm.at[iidxv], indv, sem))
    for c in copies:
        c.wait()

    gate = jnp.where(jnp.max(indv[...]) > 0.0, 1.0, 0.0)
    for g in range(_GROUPS):
        sl = pl.ds(g * 16, 16)
        valv[sl] = valv[sl] * gate
    pltpu.sync_copy(valv, sat_hbm.at[pl.ds(rbase, _ROWS_PER_W)])


_sc_fn = functools.partial(
    pl.kernel,
    mesh=plsc.VectorSubcoreMesh(core_axis_name="c", subcore_axis_name="s"),
    compiler_params=pltpu.CompilerParams(
        needs_layout_passes=False,
        skip_device_barrier=True,
        disable_semaphore_checks=True,
    ),
    out_type=jax.ShapeDtypeStruct((B,), jnp.float32),
    scratch_types=[
        pltpu.VMEM((_ROWS_PER_W,), jnp.int32),
        pltpu.VMEM((_ROWS_PER_W,), jnp.float32),
        pltpu.VMEM((16,), jnp.int32),
        pltpu.VMEM((16,), jnp.float32),
        pltpu.SemaphoreType.DMA,
    ],
)(_sc_body)


def kernel(z_1, z_2, dist_grade, img, given_param):
    dgp, gidx = _tc_stage(z_1, z_2, dist_grade)
    return _sc_fn(dgp.reshape(-1), gidx.reshape(-1))


# speedup vs baseline: 1.1803x; 1.0038x over previous
import functools

import jax
import jax.numpy as jnp
import numpy as np
from jax import lax
from jax.experimental import pallas as pl
from jax.experimental.pallas import tpu as pltpu
from jax.experimental.pallas import tpu_sc as plsc

RHO_NUM = 100
B = 16384
D = 11

_DGW = 128
_ZERO_COL = 127
_IND_COL = 126
_TC_BLK = 4096
_TC_GRID = B // _TC_BLK
_ROWS_PER_W = B // 32
_GROUPS = _ROWS_PER_W // 16


def _bucket_thresholds():
    c = np.float32(1.0 / RHO_NUM)
    t = np.array([np.float32(0.01 * i) for i in range(1, RHO_NUM)], np.float32)

    def bucket_id(s):
        r = np.sqrt(np.float32(s), dtype=np.float32)
        k = np.round(np.float32(r / c)).astype(np.float32)
        return int(np.sum(np.float32(k * c) >= t))

    out = np.empty(RHO_NUM - 1, np.float32)
    for j in range(1, RHO_NUM):
        lo, hi = 0, int(np.array(1e8, np.float32).view(np.uint32))
        while lo < hi:
            mid = (lo + hi) // 2
            if bucket_id(np.array(mid, np.uint32).view(np.float32)) >= j:
                hi = mid
            else:
                lo = mid + 1
        out[j - 1] = np.array(lo, np.uint32).view(np.float32)
    return out


_S_LIST = [float(v) for v in _bucket_thresholds()]


def _tc_body(z1_ref, z2_ref, dg_ref, dgp_ref, gidx_ref, s_scr, m_scr):
    i = pl.program_id(0)
    dx = z1_ref[:, D - 2] - z2_ref[:, D - 2]
    dy = z1_ref[:, D - 1] - z2_ref[:, D - 1]
    s = dx * dx + dy * dy
    mask = (z1_ref[:, 0] > 0.0) & (z2_ref[:, 0] > 0.0)
    sl = _TC_BLK // 128
    s_scr[...] = s.reshape(sl, 128)
    m_scr[...] = jnp.where(mask, 1.0, 0.0).reshape(sl, 128)
    s8 = s_scr[...]
    mask8 = m_scr[...] > 0.0
    bid8 = jnp.zeros((sl, 128), jnp.int32)
    for thr in _S_LIST:
        bid8 = bid8 + (s8 >= thr).astype(jnp.int32)
    col8 = jnp.where(mask8, bid8, _ZERO_COL)
    n8 = (lax.broadcasted_iota(jnp.int32, (sl, 128), 0) * 128
          + lax.broadcasted_iota(jnp.int32, (sl, 128), 1))
    gidx_ref[...] = ((i * _TC_BLK + n8) * _DGW + col8).reshape(1, sl, 128)

    ne = jnp.max(jnp.where(z1_ref[...] != z2_ref[...], 1.0, 0.0))
    dgp_ref[:, :RHO_NUM] = dg_ref[...]
    dgp_ref[:, RHO_NUM:] = jnp.zeros((_TC_BLK, _DGW - RHO_NUM), jnp.float32)
    dgp_ref[:, _IND_COL:_IND_COL + 1] = jnp.full((_TC_BLK, 1), ne,
                                                 jnp.float32)


def _tc_stage(z_1, z_2, dist_grade):
    return pl.pallas_call(
        _tc_body,
        grid=(_TC_GRID,),
        in_specs=[
            pl.BlockSpec((_TC_BLK, D), lambda i: (i, 0)),
            pl.BlockSpec((_TC_BLK, D), lambda i: (i, 0)),
            pl.BlockSpec((_TC_BLK, RHO_NUM), lambda i: (i, 0)),
        ],
        out_specs=[
            pl.BlockSpec((_TC_BLK, _DGW), lambda i: (i, 0)),
            pl.BlockSpec((1, _TC_BLK // 128, 128), lambda i: (i, 0, 0)),
        ],
        out_shape=[
            jax.ShapeDtypeStruct((B, _DGW), jnp.float32),
            jax.ShapeDtypeStruct((_TC_GRID, _TC_BLK // 128, 128), jnp.int32),
        ],
        scratch_shapes=[
            pltpu.VMEM((_TC_BLK // 128, 128), jnp.float32),
            pltpu.VMEM((_TC_BLK // 128, 128), jnp.float32),
        ],
    )(z_1, z_2, dist_grade)


def _sc_body(dgp_hbm, gidx_hbm, sat_hbm, idxv, valv, iidxv, indv, sem):
    wid = lax.axis_index("s") * 2 + lax.axis_index("c")
    rbase = wid * _ROWS_PER_W

    pltpu.sync_copy(gidx_hbm.at[pl.ds(rbase, _ROWS_PER_W)], idxv)
    iidxv[...] = (jnp.minimum(lax.iota(jnp.int32, 16), _TC_GRID - 1)
                  * (_TC_BLK * _DGW) + _IND_COL)

    copies = [
        pltpu.async_copy(
            dgp_hbm.at[idxv.at[pl.ds(i * 128, 128)]],
            valv.at[pl.ds(i * 128, 128)],
            sem,
        )
        for i in range(_ROWS_PER_W // 128)
    ]
    copies.append(pltpu.async_copy(dgp_hbm.at[iidxv], indv, sem))
    for c in copies:
        c.wait()

    gate = jnp.where(jnp.max(indv[...]) > 0.0, 1.0, 0.0)
    for g in range(_GROUPS):
        sl = pl.ds(g * 16, 16)
        valv[sl] = valv[sl] * gate
    pltpu.sync_copy(valv, sat_hbm.at[pl.ds(rbase, _ROWS_PER_W)])


_sc_fn = functools.partial(
    pl.kernel,
    mesh=plsc.VectorSubcoreMesh(core_axis_name="c", subcore_axis_name="s"),
    compiler_params=pltpu.CompilerParams(
        needs_layout_passes=False,
        skip_device_barrier=True,
        disable_semaphore_checks=True,
    ),
    out_type=jax.ShapeDtypeStruct((B,), jnp.float32),
    scratch_types=[
        pltpu.VMEM((_ROWS_PER_W,), jnp.int32),
        pltpu.VMEM((_ROWS_PER_W,), jnp.float32),
        pltpu.VMEM((16,), jnp.int32),
        pltpu.VMEM((16,), jnp.float32),
        pltpu.SemaphoreType.DMA,
    ],
)(_sc_body)


def kernel(z_1, z_2, dist_grade, img, given_param):
    dgp, gidx = _tc_stage(z_1, z_2, dist_grade)
    return _sc_fn(dgp.reshape(-1), gidx.reshape(-1))
